# baseline (speedup 1.0000x reference)
import jax
import jax.numpy as jnp
from jax.experimental import pallas as pl
from jax.experimental.pallas import tpu as pltpu


def kernel(x, i, t, pad_lengths, decoder_embedding_weight):
    s, b, c = x.shape
    dt, _ = decoder_embedding_weight.shape
    dims = i.shape[2]
    bd = b * dims

    CX = 8
    xrows = s // CX
    RD = 128
    nrep = dt // RD
    RB = 128
    nb = dt // RB

    t2 = t.reshape(1, b)
    pad2 = pad_lengths.reshape(1, b)
    i2 = i.reshape(s, bd)

    def body(w_hbm, x_hbm, i_hbm, t_ref, pad_ref, xo_hbm, io_hbm, po_ref,
             w_v, bufs, di_v, sem_x, sem_i, sem_di, sem_w, sem_dx):
        for k in range(CX):
            pltpu.make_async_copy(
                x_hbm.at[pl.ds(k * xrows, xrows)],
                xo_hbm.at[pl.ds(dt + k * xrows, xrows)],
                sem_x.at[k]).start()
        pltpu.make_async_copy(i_hbm, io_hbm.at[pl.ds(dt, s)], sem_i).start()
        pltpu.make_async_copy(w_hbm, w_v, sem_w).start()

        lane = jax.lax.broadcasted_iota(jnp.int32, (1, bd), 1)
        tv = jnp.repeat(t_ref[...], dims, axis=1)
        row = jnp.where(lane % dims == 0, 1,
                        jnp.where(lane % dims == 1, tv, -1))
        di_v[...] = jnp.broadcast_to(row, (RD, bd))
        for r in range(nrep):
            pltpu.make_async_copy(
                di_v, io_hbm.at[pl.ds(r * RD, RD)], sem_di.at[r]).start()

        po_ref[...] = pad_ref[...] + dt

        pltpu.make_async_copy(w_hbm, w_v, sem_w).wait()
        for p in range(nb):
            buf = bufs.at[p % 2]
            if p >= 2:
                pltpu.make_async_copy(
                    bufs.at[(p - 2) % 2],
                    xo_hbm.at[pl.ds((p - 2) * RB, RB)],
                    sem_dx.at[p - 2]).wait()
            buf[...] = jnp.broadcast_to(
                w_v[pl.ds(p * RB, RB), :][:, None, :], (RB, b, c))
            pltpu.make_async_copy(
                buf, xo_hbm.at[pl.ds(p * RB, RB)], sem_dx.at[p]).start()

        for p in range(max(nb - 2, 0), nb):
            pltpu.make_async_copy(
                bufs.at[p % 2], xo_hbm.at[pl.ds(p * RB, RB)],
                sem_dx.at[p]).wait()
        for r in range(nrep):
            pltpu.make_async_copy(
                di_v, io_hbm.at[pl.ds(r * RD, RD)], sem_di.at[r]).wait()
        pltpu.make_async_copy(i_hbm, io_hbm.at[pl.ds(dt, s)], sem_i).wait()
        for k in range(CX):
            pltpu.make_async_copy(
                x_hbm.at[pl.ds(k * xrows, xrows)],
                xo_hbm.at[pl.ds(dt + k * xrows, xrows)],
                sem_x.at[k]).wait()

    vmem = pltpu.MemorySpace.VMEM
    xo, io, po = pl.pallas_call(
        body,
        in_specs=[
            pl.BlockSpec(memory_space=pl.ANY),
            pl.BlockSpec(memory_space=pl.ANY),
            pl.BlockSpec(memory_space=pl.ANY),
            pl.BlockSpec(memory_space=vmem),
            pl.BlockSpec(memory_space=vmem),
        ],
        out_specs=[
            pl.BlockSpec(memory_space=pl.ANY),
            pl.BlockSpec(memory_space=pl.ANY),
            pl.BlockSpec(memory_space=vmem),
        ],
        out_shape=[
            jax.ShapeDtypeStruct((dt + s, b, c), x.dtype),
            jax.ShapeDtypeStruct((dt + s, bd), i.dtype),
            jax.ShapeDtypeStruct((1, b), pad_lengths.dtype),
        ],
        scratch_shapes=[
            pltpu.VMEM((dt, c), x.dtype),
            pltpu.VMEM((2, RB, b, c), x.dtype),
            pltpu.VMEM((RD, bd), i.dtype),
            pltpu.SemaphoreType.DMA((CX,)),
            pltpu.SemaphoreType.DMA,
            pltpu.SemaphoreType.DMA((nrep,)),
            pltpu.SemaphoreType.DMA,
            pltpu.SemaphoreType.DMA((nb,)),
        ],
    )(decoder_embedding_weight, x, i2, t2, pad2)
    return xo, io.reshape(dt + s, b, dims), po.reshape(b)


# speedup vs baseline: 30.8442x; 30.8442x over previous
import jax
import jax.numpy as jnp
from jax.experimental import pallas as pl


def kernel(x, i, t, pad_lengths, decoder_embedding_weight):
    s, b, c = x.shape
    dt, _ = decoder_embedding_weight.shape
    dims = i.shape[2]
    bd = b * dims
    BLK = 256
    n_dt = dt // BLK
    n_s = s // BLK
    n_total = n_dt + n_s

    t2 = t.reshape(1, b)
    pad2 = pad_lengths.reshape(1, b)
    i2 = i.reshape(s, bd)

    def body(w_ref, x_ref, i_ref, t_ref, pad_ref, xo_ref, io_ref, po_ref):
        g = pl.program_id(0)
        r = g % 3

        @pl.when(r == 2)
        def _():
            xo_ref[...] = jnp.broadcast_to(w_ref[...][:, None, :], (BLK, b, c))
            lane = jax.lax.broadcasted_iota(jnp.int32, (1, bd), 1)
            tv = jnp.repeat(t_ref[...], dims, axis=1)
            row = jnp.where(lane % dims == 0, 1,
                            jnp.where(lane % dims == 1, tv, -1))
            io_ref[...] = jnp.broadcast_to(row, (BLK, bd))

        @pl.when(r != 2)
        def _():
            xo_ref[...] = x_ref[...]
            io_ref[...] = i_ref[...]

        po_ref[...] = pad_ref[...] + dt

    def out_idx(g):
        q, r = g // 3, g % 3
        return jnp.where(r == 2, q, n_dt + 2 * q + r)

    def copy_idx(g):
        q, r = g // 3, g % 3
        return 2 * q + jnp.minimum(r, 1)

    grid = (n_total,)
    in_specs = [
        pl.BlockSpec((BLK, c), lambda g: (g // 3, 0)),
        pl.BlockSpec((BLK, b, c), lambda g: (copy_idx(g), 0, 0)),
        pl.BlockSpec((BLK, bd), lambda g: (copy_idx(g), 0)),
        pl.BlockSpec((1, b), lambda g: (0, 0)),
        pl.BlockSpec((1, b), lambda g: (0, 0)),
    ]
    out_specs = [
        pl.BlockSpec((BLK, b, c), lambda g: (out_idx(g), 0, 0)),
        pl.BlockSpec((BLK, bd), lambda g: (out_idx(g), 0)),
        pl.BlockSpec((1, b), lambda g: (0, 0)),
    ]
    out_shape = [
        jax.ShapeDtypeStruct((dt + s, b, c), x.dtype),
        jax.ShapeDtypeStruct((dt + s, bd), i.dtype),
        jax.ShapeDtypeStruct((1, b), pad_lengths.dtype),
    ]
    xo, io, po = pl.pallas_call(
        body, grid=grid, in_specs=in_specs, out_specs=out_specs,
        out_shape=out_shape,
    )(decoder_embedding_weight, x, i2, t2, pad2)
    return xo, io.reshape(dt + s, b, dims), po.reshape(b)
